# baseline (speedup 1.0000x reference)
import jax
import jax.numpy as jnp
from jax import lax
from jax.experimental import pallas as pl
from jax.experimental.pallas import tpu as pltpu
from jax.experimental.pallas import tpu_sc as plsc

NUM_EMB = 1000000
D = 32
B_TOK = 16384
S_TOK = 50

NC = 2
NS = 16
NW = NC * NS
BATCH_PER_W = B_TOK // NW

NBATCH = 4
CHUNK = NBATCH * S_TOK
N_CHUNKS = BATCH_PER_W // NBATCH
NBUF = 8
LOOKAHEAD = 6


def _body(idx_hbm, table_hbm, out_hbm, idx_v, rows_v, *sems):
    gsem = sems[:NBUF]
    ssem = sems[NBUF:]
    wid = lax.axis_index("s") * NC + lax.axis_index("c")
    b0 = wid * BATCH_PER_W
    pltpu.sync_copy(idx_hbm.at[pl.ds(b0, BATCH_PER_W)], idx_v)

    def gather(c, b):
        for j in range(NBATCH):
            pltpu.async_copy(
                table_hbm.at[idx_v.at[c * NBATCH + j]],
                rows_v.at[b, j], gsem[b])

    def gather_wait(b):
        for j in range(NBATCH):
            pltpu.make_async_copy(
                table_hbm.at[idx_v.at[j]],
                rows_v.at[b, j], gsem[b]).wait()

    def scatter(c, b):
        pltpu.async_copy(
            rows_v.at[b],
            out_hbm.at[pl.ds(b0 + c * NBATCH, NBATCH), 0:S_TOK, 0:D],
            ssem[b])

    def scatter_wait(b):
        pltpu.make_async_copy(
            rows_v.at[b],
            out_hbm.at[pl.ds(b0, NBATCH), 0:S_TOK, 0:D], ssem[b]).wait()

    for c in range(LOOKAHEAD):
        gather(c, c)

    def group(i, carry):
        for b in range(NBUF):
            c = i * NBUF + b
            gather_wait(b)
            scatter(c, b)
            c2 = c + LOOKAHEAD
            b2 = (b + LOOKAHEAD) % NBUF

            @pl.when(c2 < N_CHUNKS)
            def _():
                @pl.when(c2 >= NBUF)
                def _():
                    scatter_wait(b2)
                gather(c2, b2)
        return carry

    lax.fori_loop(0, N_CHUNKS // NBUF, group, 0)

    for b in range(NBUF):
        scatter_wait(b)


S_PAD = 56
D_PAD = 128

DET_CHUNK = 40
DET_CNT = 7840
DET_CNT_LAST = 250000 - 31 * DET_CNT
DET_ITERS = DET_CNT // DET_CHUNK


def _detile_body(w_hbm, w3_hbm, stripes_v, dense_v):
    wid = lax.axis_index("s") * NC + lax.axis_index("c")
    start = wid * DET_CNT
    n_chunks = jnp.where(wid == NW - 1, DET_CNT_LAST // DET_CHUNK, DET_ITERS)

    def chunk_body(c, carry):
        @pl.when(c < n_chunks)
        def _():
            r3 = start + c * DET_CHUNK
            rw = r3 * 4
            pltpu.sync_copy(
                w_hbm.at[pl.ds(rw, 4 * DET_CHUNK)], stripes_v)

            def pack(r4, carry2):
                for q in range(4):
                    for k in range(2):
                        dense_v[r4, pl.ds(32 * q + 16 * k, 16)] = (
                            stripes_v[4 * r4 + q, pl.ds(16 * k, 16)])
                return carry2

            lax.fori_loop(0, DET_CHUNK, pack, 0)
            pltpu.sync_copy(dense_v, w3_hbm.at[pl.ds(r3, DET_CHUNK)])
        return carry

    lax.fori_loop(0, DET_ITERS, chunk_body, 0)


def _mesh():
    return plsc.VectorSubcoreMesh(
        core_axis_name="c", subcore_axis_name="s", num_cores=NC,
        num_subcores=NS)


@jax.jit
def _gather(idx, weight):
    det = pl.kernel(
        _detile_body,
        out_type=jax.ShapeDtypeStruct((NUM_EMB // 4, D_PAD), jnp.float32),
        mesh=_mesh(),
        scratch_types=[
            pltpu.VMEM((4 * DET_CHUNK, D), jnp.float32),
            pltpu.VMEM((DET_CHUNK, D_PAD), jnp.float32),
        ],
    )
    w_lin = det(weight).reshape(NUM_EMB, D)
    f = pl.kernel(
        _body,
        out_type=jax.ShapeDtypeStruct((B_TOK, S_PAD, D_PAD), jnp.float32),
        mesh=_mesh(),
        scratch_types=[
            pltpu.VMEM((BATCH_PER_W, S_TOK), jnp.int32),
            pltpu.VMEM((NBUF, NBATCH, S_TOK, D), jnp.float32),
        ] + [pltpu.SemaphoreType.DMA] * (2 * NBUF),
        compiler_params=pltpu.CompilerParams(use_tc_tiling_on_sc=False),
    )
    zpad = f(idx, w_lin)
    return lax.slice(zpad, (0, 0, 0), (B_TOK, S_TOK, D))


def kernel(token_ids, weight):
    return _gather(token_ids.astype(jnp.int32), weight)
